# Initial kernel scaffold; baseline (speedup 1.0000x reference)
import jax
import jax.numpy as jnp
from jax.experimental import pallas as pl


def kernel(xyz, features, Wq, bq, Wk, bk, Wv, bv, P1, p1, P2, p2, F1, f1, F2, f2, g1, be1, g2, be2):
    raise NotImplementedError("write your pallas kernel here")


# speedup vs baseline: 7.4725x; 7.4725x over previous
import functools

import jax
import jax.numpy as jnp
from jax import lax
from jax.experimental import pallas as pl
from jax.experimental.pallas import tpu as pltpu
from jax.experimental.pallas import tpu_sc as plsc

B, N, D, DO, H, K = 4, 2048, 256, 256, 4, 16
HD = DO // H

R1 = 256
R2 = 512
R4 = 128
RK = R4 * K


def _knn_body(xr_ref, xat_ref, idx_ref):
    b = pl.program_id(0)
    xr = xr_ref[0]
    xat = xat_ref[0]
    x2r = jnp.sum(xr * xr, axis=1, keepdims=True)
    x2a = jnp.sum(xat * xat, axis=0, keepdims=True)
    d2 = x2r + x2a - 2.0 * jnp.dot(xr, xat, preferred_element_type=jnp.float32)
    iota = lax.broadcasted_iota(jnp.int32, (R1, N), 1)
    cols = []
    for _ in range(K):
        m = jnp.min(d2, axis=1, keepdims=True)
        sel = jnp.where(d2 == m, iota, N)
        idx = jnp.min(sel, axis=1, keepdims=True)
        cols.append(idx)
        d2 = jnp.where(iota == idx, jnp.float32(jnp.inf), d2)
    idx_ref[0] = jnp.concatenate(cols, axis=1) + b * N


def _knn(xyz8, xyz8t):
    return pl.pallas_call(
        _knn_body,
        grid=(B, N // R1),
        in_specs=[
            pl.BlockSpec((1, R1, 8), lambda b, i: (b, i, 0)),
            pl.BlockSpec((1, 8, N), lambda b, i: (b, 0, 0)),
        ],
        out_specs=pl.BlockSpec((1, R1, K), lambda b, i: (b, i, 0)),
        out_shape=jax.ShapeDtypeStruct((B, N, K), jnp.int32),
    )(xyz8, xyz8t)


def _proj_body(f_ref, wq_ref, bq_ref, wk_ref, bk_ref, wv_ref, bv_ref,
               g1_ref, be1_ref, q_ref, kv_ref, skip_ref):
    f = f_ref[0]
    q = jnp.dot(f, wq_ref[...], preferred_element_type=jnp.float32) + bq_ref[...]
    kk = jnp.dot(f, wk_ref[...], preferred_element_type=jnp.float32) + bk_ref[...]
    vv = jnp.dot(f, wv_ref[...], preferred_element_type=jnp.float32) + bv_ref[...]
    mu = jnp.mean(f, axis=1, keepdims=True)
    var = jnp.mean((f - mu) ** 2, axis=1, keepdims=True)
    ln = (f - mu) / jnp.sqrt(var + 1e-5) * g1_ref[...] + be1_ref[...]
    skip = jnp.dot(ln, wq_ref[...], preferred_element_type=jnp.float32) + bq_ref[...]
    q_ref[0] = q
    kv_ref[0] = jnp.concatenate([kk, vv], axis=1)
    skip_ref[0] = skip


def _proj(features, Wq, bq, Wk, bk, Wv, bv, g1, be1):
    full = lambda shape: pl.BlockSpec(shape, lambda b, i: tuple(0 for _ in shape))
    return pl.pallas_call(
        _proj_body,
        grid=(B, N // R2),
        in_specs=[
            pl.BlockSpec((1, R2, D), lambda b, i: (b, i, 0)),
            full((D, DO)), full((1, DO)), full((D, DO)), full((1, DO)),
            full((D, DO)), full((1, DO)), full((1, D)), full((1, D)),
        ],
        out_specs=[
            pl.BlockSpec((1, R2, DO), lambda b, i: (b, i, 0)),
            pl.BlockSpec((1, R2, 2 * DO), lambda b, i: (b, i, 0)),
            pl.BlockSpec((1, R2, DO), lambda b, i: (b, i, 0)),
        ],
        out_shape=[
            jax.ShapeDtypeStruct((B, N, DO), jnp.float32),
            jax.ShapeDtypeStruct((B, N, 2 * DO), jnp.float32),
            jax.ShapeDtypeStruct((B, N, DO), jnp.float32),
        ],
    )(features, Wq, bq, Wk, bk, Wv, bv, g1, be1)


ROWS = B * N * K
_info = plsc.get_sparse_core_info()
NC, NS = _info.num_cores, _info.num_subcores
NW = NC * NS
PER_W = ROWS // NW
CHUNK = 128
NCH = PER_W // CHUNK


@functools.partial(
    pl.kernel,
    mesh=plsc.VectorSubcoreMesh(core_axis_name="c", subcore_axis_name="s"),
    out_type=[
        jax.ShapeDtypeStruct((ROWS, 2 * DO), jnp.float32),
        jax.ShapeDtypeStruct((ROWS, 128), jnp.float32),
    ],
    scratch_types=[
        pltpu.VMEM((CHUNK,), jnp.int32),
        pltpu.VMEM((CHUNK, 2 * DO), jnp.float32),
        pltpu.VMEM((CHUNK, 128), jnp.float32),
        pltpu.SemaphoreType.DMA,
        pltpu.SemaphoreType.DMA,
    ],
)
def _sc_gather(idx_hbm, kv_hbm, xyz_hbm, okv_hbm, oxyz_hbm,
               idxc, rows_v, xrows_v, sem, sem2):
    wid = lax.axis_index("s") * NC + lax.axis_index("c")
    base = wid * PER_W

    def body(i, carry):
        off = base + i * CHUNK
        pltpu.sync_copy(idx_hbm.at[pl.ds(off, CHUNK)], idxc)
        cp1 = pltpu.async_copy(kv_hbm.at[idxc], rows_v, sem)
        cp2 = pltpu.async_copy(xyz_hbm.at[idxc], xrows_v, sem2)
        cp1.wait()
        cp2.wait()
        pltpu.sync_copy(rows_v, okv_hbm.at[pl.ds(off, CHUNK)])
        pltpu.sync_copy(xrows_v, oxyz_hbm.at[pl.ds(off, CHUNK)])
        return carry

    lax.fori_loop(0, NCH, body, 0)


def _attn_body(q_ref, kv_ref, xg_ref, xc_ref, skip_ref,
               p1w_ref, p1b_ref, p2w_ref, p2b_ref,
               f1w_ref, f1b_ref, f2w_ref, f2b_ref, g2_ref, be2_ref,
               out_ref, attn_ref):
    q2 = q_ref[0]
    kv = kv_ref[0]
    xg = xg_ref[0]
    xc = xc_ref[0]

    rel = xg.reshape(R4, K, 128) - xc[:, None, :]
    relf = rel.reshape(RK, 128)
    h1 = jnp.maximum(jnp.dot(relf, p1w_ref[...], preferred_element_type=jnp.float32)
                     + p1b_ref[...], 0.0)
    pos = jnp.dot(h1, p2w_ref[...], preferred_element_type=jnp.float32) + p2b_ref[...]
    pos3 = pos.reshape(R4, K, HD)

    outs, attns = [], []
    for h in range(H):
        qh = q2[:, HD * h:HD * (h + 1)]
        kh = kv[:, HD * h:HD * (h + 1)].reshape(R4, K, HD)
        vh = kv[:, DO + HD * h:DO + HD * (h + 1)].reshape(R4, K, HD)
        lg = jnp.sum(qh[:, None, :] * (kh + pos3), axis=2) / (HD ** 0.5)
        mx = jnp.max(lg, axis=1, keepdims=True)
        e = jnp.exp(lg - mx)
        aw = e / jnp.sum(e, axis=1, keepdims=True)
        attns.append(aw)
        outs.append(jnp.sum(aw[:, :, None] * (vh + pos3), axis=1))

    o = jnp.concatenate(outs, axis=1)
    y = jnp.maximum(jnp.dot(o, f1w_ref[...], preferred_element_type=jnp.float32)
                    + f1b_ref[...], 0.0)
    y = jnp.dot(y, f2w_ref[...], preferred_element_type=jnp.float32) + f2b_ref[...]
    mu = jnp.mean(y, axis=1, keepdims=True)
    var = jnp.mean((y - mu) ** 2, axis=1, keepdims=True)
    y = (y - mu) / jnp.sqrt(var + 1e-5) * g2_ref[...] + be2_ref[...]
    out_ref[0] = y + skip_ref[0]
    attn_ref[0] = jnp.stack(attns, axis=0)


def _attn(q, kvg, xyzg, xyz16, skip, P1p, p1, P2, p2, F1, f1, F2, f2, g2, be2):
    full = lambda shape: pl.BlockSpec(shape, lambda b, i: tuple(0 for _ in shape))
    return pl.pallas_call(
        _attn_body,
        grid=(B, N // R4),
        in_specs=[
            pl.BlockSpec((1, R4, DO), lambda b, i: (b, i, 0)),
            pl.BlockSpec((1, RK, 2 * DO), lambda b, i: (b, i, 0)),
            pl.BlockSpec((1, RK, 128), lambda b, i: (b, i, 0)),
            pl.BlockSpec((1, R4, 128), lambda b, i: (b, i, 0)),
            pl.BlockSpec((1, R4, DO), lambda b, i: (b, i, 0)),
            full((128, HD)), full((1, HD)), full((HD, HD)), full((1, HD)),
            full((D, DO)), full((1, DO)), full((DO, DO)), full((1, DO)),
            full((1, DO)), full((1, DO)),
        ],
        out_specs=[
            pl.BlockSpec((1, R4, DO), lambda b, i: (b, i, 0)),
            pl.BlockSpec((1, H, R4, K), lambda b, i: (b, 0, i, 0)),
        ],
        out_shape=[
            jax.ShapeDtypeStruct((B, N, DO), jnp.float32),
            jax.ShapeDtypeStruct((B, H, N, K), jnp.float32),
        ],
    )(q, kvg, xyzg, xyz16, skip, P1p, p1, P2, p2, F1, f1, F2, f2, g2, be2)


def kernel(xyz, features, Wq, bq, Wk, bk, Wv, bv, P1, p1, P2, p2,
           F1, f1, F2, f2, g1, be1, g2, be2):
    xyz8 = jnp.pad(xyz, ((0, 0), (0, 0), (0, 5)))
    xyz8t = jnp.transpose(xyz8, (0, 2, 1))
    xyz128 = jnp.pad(xyz, ((0, 0), (0, 0), (0, 125)))
    P1p = jnp.pad(P1, ((0, 125), (0, 0)))

    idxg = _knn(xyz8, xyz8t)
    q, kv, skip = _proj(features, Wq, bq.reshape(1, DO), Wk, bk.reshape(1, DO),
                        Wv, bv.reshape(1, DO), g1.reshape(1, D), be1.reshape(1, D))

    kvg, xyzg = _sc_gather(idxg.reshape(ROWS),
                           kv.reshape(B * N, 2 * DO),
                           xyz128.reshape(B * N, 128))

    out, attn = _attn(q, kvg.reshape(B, N * K, 2 * DO), xyzg.reshape(B, N * K, 128),
                      xyz128, skip, P1p, p1.reshape(1, HD), P2, p2.reshape(1, HD),
                      F1, f1.reshape(1, DO), F2, f2.reshape(1, DO),
                      g2.reshape(1, DO), be2.reshape(1, DO))
    return (out, attn)
